# baseline (speedup 1.0000x reference)
import functools

import jax
import jax.numpy as jnp
from jax import lax
from jax.experimental import pallas as pl
from jax.experimental.pallas import tpu as pltpu
from jax.experimental.pallas import tpu_sc as plsc

_H = 512
_W = 512
_B = 16
_BIG = _H + _W
_NJ = _W // 16

_NSC = 4
_NTC = _B - _NSC
_SPI = 32 // _NSC
_ROWS_PER_SUB = _NSC * _H // 32
_CHUNK = 32
_NCH = _ROWS_PER_SUB // _CHUNK


def _lane_allreduce(v, op, lane):
    for sh in (8, 4, 2, 1):
        idx = jnp.bitwise_xor(lane, sh)
        pv = lax.gather(
            v, idx[:, None],
            lax.GatherDimensionNumbers(offset_dims=(),
                                       collapsed_slice_dims=(0,),
                                       start_index_map=(0,)),
            slice_sizes=(1,),
            mode=lax.GatherScatterMode.PROMISE_IN_BOUNDS)
        v = op(v, pv)
    return v


def _lane_pack(vals, lane):
    out = jnp.zeros((16,), vals[0].dtype)
    for i, v in enumerate(vals):
        out = jnp.where(lane == i, v, out)
    return out


def _sc_body(mask_hbm, out_hbm, part_hbm, buf0, buf1, colcnt, tmp8, tb, tmpf,
             sem0, sem1):
    c = lax.axis_index("c")
    s = lax.axis_index("s")
    wid = c * 16 + s
    b = wid // _SPI
    p = wid % _SPI
    row_base = b * _H + p * _ROWS_PER_SUB
    y_base = p * _ROWS_PER_SUB

    zero16 = jnp.zeros((16,), jnp.int32)
    one_i = jnp.int32(1)
    zero_i = jnp.int32(0)
    for j in range(_NJ):
        colcnt[pl.ds(j * 16, 16)] = zero16

    bufs = (buf0, buf1)
    sems = (sem0, sem1)
    cps = [None, None]
    cps[0] = pltpu.async_copy(mask_hbm.at[pl.ds(row_base, _CHUNK)], buf0, sem0)

    big_v = jnp.full((16,), _BIG, jnp.int32)
    neg1_v = jnp.full((16,), -1, jnp.int32)
    carry = (zero16, big_v, neg1_v)

    for ch in range(_NCH):
        cur = ch % 2
        nxt = (ch + 1) % 2
        if ch + 1 < _NCH:
            cps[nxt] = pltpu.async_copy(
                mask_hbm.at[pl.ds(row_base + (ch + 1) * _CHUNK, _CHUNK)],
                bufs[nxt], sems[nxt])
        cps[cur].wait()
        buf = bufs[cur]
        y0 = y_base + ch * _CHUNK

        def row_body(r, carry, buf=buf, y0=y0):
            sum_y, y_min, y_max = carry

            @plsc.parallel_loop(0, _W, step=16, unroll=8, carry=zero16)
            def rowcnt(off, rc):
                v = buf[r, pl.ds(off, 16)]
                sel = jnp.where(v > 0.5, one_i, zero_i)
                plsc.addupdate(colcnt.at[pl.ds(off, 16)], sel)
                return rc + sel

            yv = jnp.broadcast_to(y0 + r, (16,)).astype(jnp.int32)
            any_ = rowcnt > 0
            sum_y = sum_y + yv * rowcnt
            y_min = jnp.minimum(y_min, jnp.where(any_, yv, big_v))
            y_max = jnp.where(any_, yv, y_max)
            return (sum_y, y_min, y_max)

        carry = lax.fori_loop(0, _CHUNK, row_body, carry)

    sum_y_v, y_min_v, y_max_v = carry

    lane = lax.iota(jnp.int32, 16)
    cnt_v = zero16
    sum_x_v = zero16
    x_min_v = big_v
    x_max_v = neg1_v
    for j in range(_NJ):
        cc = colcnt[pl.ds(j * 16, 16)]
        xv = lane + (j * 16)
        cnt_v = cnt_v + cc
        sum_x_v = sum_x_v + xv * cc
        colany = cc > 0
        x_min_v = jnp.minimum(x_min_v, jnp.where(colany, xv, big_v))
        x_max_v = jnp.maximum(x_max_v, jnp.where(colany, xv, neg1_v))

    cnt_r = _lane_allreduce(cnt_v, jnp.add, lane)
    sy_r = _lane_allreduce(sum_y_v, jnp.add, lane)
    sx_r = _lane_allreduce(sum_x_v, jnp.add, lane)
    ymin_r = _lane_allreduce(y_min_v, jnp.minimum, lane)
    ymax_r = _lane_allreduce(y_max_v, jnp.maximum, lane)
    xmin_r = _lane_allreduce(x_min_v, jnp.minimum, lane)
    xmax_r = _lane_allreduce(x_max_v, jnp.maximum, lane)

    tmp8[0, :] = cnt_r
    tmp8[1, :] = sy_r
    tmp8[2, :] = sx_r
    tmp8[3, :] = ymin_r
    tmp8[4, :] = ymax_r
    tmp8[5, :] = xmin_r
    tmp8[6, :] = xmax_r
    tmp8[7, :] = zero16

    pltpu.sync_copy(tmp8, part_hbm.at[wid])
    plsc.subcore_barrier()

    @pl.when(p == 0)
    def _():
        count_v, sy_v, sx_v = cnt_r, sy_r, sx_r
        ymin_v, ymax_v, xmin_v, xmax_v = ymin_r, ymax_r, xmin_r, xmax_r
        pltpu.sync_copy(part_hbm.at[pl.ds(wid + 1, _SPI - 1)], tb)
        for q in range(_SPI - 1):
            count_v = count_v + tb[q, 0, :]
            sy_v = sy_v + tb[q, 1, :]
            sx_v = sx_v + tb[q, 2, :]
            ymin_v = jnp.minimum(ymin_v, tb[q, 3, :])
            ymax_v = jnp.maximum(ymax_v, tb[q, 4, :])
            xmin_v = jnp.minimum(xmin_v, tb[q, 5, :])
            xmax_v = jnp.maximum(xmax_v, tb[q, 6, :])

        height = ymax_v - ymin_v + 1
        width = xmax_v - xmin_v + 1
        size = jnp.maximum(height, width)
        cy_i = lax.shift_right_arithmetic(ymin_v + ymax_v, 1)
        cx_i = lax.shift_right_arithmetic(xmin_v + xmax_v, 1)
        half_sz = lax.shift_right_arithmetic(size, 1)
        y1 = jnp.maximum(0, cy_i - half_sz)
        x1 = jnp.maximum(0, cx_i - half_sz)
        y2 = jnp.minimum(_H, cy_i + half_sz)
        x2 = jnp.minimum(_W, cx_i + half_sz)

        empty = count_v == 0
        denom = jnp.maximum(count_v.astype(jnp.float32), 1.0)
        cy_f = sy_v.astype(jnp.float32) / denom
        cx_f = sx_v.astype(jnp.float32) / denom
        cy_f = jnp.where(empty, jnp.float32(_H // 2), cy_f)
        cx_f = jnp.where(empty, jnp.float32(_W // 2), cx_f)
        size_f = jnp.where(empty, min(_H, _W) // 2, size).astype(jnp.float32)
        y1_f = jnp.where(empty, _H // 4, y1).astype(jnp.float32)
        x1_f = jnp.where(empty, _W // 4, x1).astype(jnp.float32)
        y2_f = jnp.where(empty, 3 * _H // 4, y2).astype(jnp.float32)
        x2_f = jnp.where(empty, 3 * _W // 4, x2).astype(jnp.float32)

        outv = _lane_pack([cy_f, cx_f, size_f, y1_f, x1_f, y2_f, x2_f], lane)
        tmpf[...] = outv
        pltpu.sync_copy(tmpf, out_hbm.at[b])


_sc_kernel = functools.partial(
    pl.kernel,
    mesh=plsc.VectorSubcoreMesh(core_axis_name="c", subcore_axis_name="s"),
    out_type=(
        jax.ShapeDtypeStruct((_NSC, 16), jnp.float32),
        jax.ShapeDtypeStruct((32, 8, 16), jnp.int32),
    ),
    scratch_types=[
        pltpu.VMEM((_CHUNK, _W), jnp.float32),
        pltpu.VMEM((_CHUNK, _W), jnp.float32),
        pltpu.VMEM((_W,), jnp.int32),
        pltpu.VMEM((8, 16), jnp.int32),
        pltpu.VMEM((_SPI - 1, 8, 16), jnp.int32),
        pltpu.VMEM((16,), jnp.float32),
        pltpu.SemaphoreType.DMA,
        pltpu.SemaphoreType.DMA,
    ],
)(_sc_body)


_HC = 128
_NHC = _H // _HC


def _tc_body(mask_ref, stats_ref, colcnt_ref):
    hc = pl.program_id(1)
    v = mask_ref[0]
    mb = (v > 0.5).astype(jnp.bfloat16)

    ones_w = jnp.ones((_W, 1), jnp.bfloat16)
    rc = jax.lax.dot_general(
        mb, ones_w, (((1,), (0,)), ((), ())),
        preferred_element_type=jnp.float32)
    colsum = jnp.sum(mb.astype(jnp.float32), axis=0).reshape(1, _W)

    @pl.when(hc == 0)
    def _():
        colcnt_ref[...] = colsum

    @pl.when(hc > 0)
    def _():
        colcnt_ref[...] = colcnt_ref[...] + colsum

    yidx = (lax.broadcasted_iota(jnp.int32, (_HC, 1), 0)
            .astype(jnp.float32) + hc.astype(jnp.float32) * _HC)
    bigf = jnp.float32(_BIG)
    part_cnt = jnp.sum(rc)
    part_sy = jnp.sum(yidx * rc)
    part_ymin = jnp.min(jnp.where(rc > 0.0, yidx, bigf))
    part_ymax = jnp.max(jnp.where(rc > 0.0, yidx, -1.0))

    @pl.when(hc == 0)
    def _():
        stats_ref[0, 0, 0] = part_cnt
        stats_ref[0, 0, 1] = part_sy
        stats_ref[0, 0, 2] = part_ymin
        stats_ref[0, 0, 3] = part_ymax

    @pl.when(hc > 0)
    def _():
        stats_ref[0, 0, 0] = stats_ref[0, 0, 0] + part_cnt
        stats_ref[0, 0, 1] = stats_ref[0, 0, 1] + part_sy
        stats_ref[0, 0, 2] = jnp.minimum(stats_ref[0, 0, 2], part_ymin)
        stats_ref[0, 0, 3] = jnp.maximum(stats_ref[0, 0, 3], part_ymax)

    @pl.when(hc == _NHC - 1)
    def _():
        count = stats_ref[0, 0, 0]
        sum_y = stats_ref[0, 0, 1]
        y_min = stats_ref[0, 0, 2].astype(jnp.int32)
        y_max = stats_ref[0, 0, 3].astype(jnp.int32)

        colcount = colcnt_ref[0, :]
        xidx = lax.iota(jnp.int32, _W).astype(jnp.float32)
        sum_x = jnp.sum(xidx * colcount)
        x_min = jnp.min(jnp.where(colcount > 0.0, xidx, bigf)).astype(jnp.int32)
        x_max = jnp.max(jnp.where(colcount > 0.0, xidx, -1.0)).astype(jnp.int32)

        denom = jnp.maximum(count, 1.0)
        center_y = sum_y / denom
        center_x = sum_x / denom

        height = y_max - y_min + 1
        width = x_max - x_min + 1
        size = jnp.maximum(height, width)

        cy_i = (y_min + y_max) // 2
        cx_i = (x_min + x_max) // 2
        half = size // 2
        y1 = jnp.maximum(0, cy_i - half)
        x1 = jnp.maximum(0, cx_i - half)
        y2 = jnp.minimum(_H, cy_i + half)
        x2 = jnp.minimum(_W, cx_i + half)

        empty = count == 0.0
        center_y = jnp.where(empty, jnp.float32(_H // 2), center_y)
        center_x = jnp.where(empty, jnp.float32(_W // 2), center_x)
        size_out = jnp.where(empty, jnp.int32(min(_H, _W) // 2), size)
        y1 = jnp.where(empty, jnp.int32(_H // 4), y1)
        x1 = jnp.where(empty, jnp.int32(_W // 4), x1)
        y2 = jnp.where(empty, jnp.int32(3 * _H // 4), y2)
        x2 = jnp.where(empty, jnp.int32(3 * _W // 4), x2)

        stats_ref[0, 0, 0] = center_y
        stats_ref[0, 0, 1] = center_x
        stats_ref[0, 0, 2] = size_out.astype(jnp.float32)
        stats_ref[0, 0, 3] = y1.astype(jnp.float32)
        stats_ref[0, 0, 4] = x1.astype(jnp.float32)
        stats_ref[0, 0, 5] = y2.astype(jnp.float32)
        stats_ref[0, 0, 6] = x2.astype(jnp.float32)
        for k in range(7, 16):
            stats_ref[0, 0, k] = jnp.float32(0)


def _tc_kernel(m3):
    return pl.pallas_call(
        _tc_body,
        grid=(_NTC, _NHC),
        in_specs=[pl.BlockSpec((1, _HC, _W), lambda b, hc: (b + _NSC, hc, 0))],
        out_specs=[
            pl.BlockSpec((1, 1, 16), lambda b, hc: (b, 0, 0),
                         memory_space=pltpu.SMEM),
        ],
        out_shape=[
            jax.ShapeDtypeStruct((_NTC, 1, 16), jnp.float32),
        ],
        scratch_shapes=[
            pltpu.VMEM((1, _W), jnp.float32),
        ],
    )(m3)


@jax.jit
def kernel(mask):
    B = mask.shape[0]
    m3 = mask.reshape(B, _H, _W)
    m2 = m3.reshape(B * _H, _W)

    sc_stats, _ = _sc_kernel(m2)
    (tc_stats,) = _tc_kernel(m3)

    stats = jnp.concatenate([sc_stats, tc_stats.reshape(_NTC, 16)], axis=0)
    centers = stats[:, :2]
    sizes = stats[:, 2].astype(jnp.int32)
    bboxes = stats[:, 3:7].astype(jnp.int32)
    return centers, sizes, bboxes


# speedup vs baseline: 1.4719x; 1.4719x over previous
import functools

import jax
import jax.numpy as jnp
from jax import lax
from jax.experimental import pallas as pl
from jax.experimental.pallas import tpu as pltpu
from jax.experimental.pallas import tpu_sc as plsc

_H = 512
_W = 512
_B = 16
_BIG = _H + _W
_NJ = _W // 16

_NSC = 4
_NTC = _B - _NSC
_SPI = 32 // _NSC
_ROWS_PER_SUB = _NSC * _H // 32
_CHUNK = 32
_NCH = _ROWS_PER_SUB // _CHUNK


def _lane_allreduce(v, op, lane):
    for sh in (8, 4, 2, 1):
        idx = jnp.bitwise_xor(lane, sh)
        pv = lax.gather(
            v, idx[:, None],
            lax.GatherDimensionNumbers(offset_dims=(),
                                       collapsed_slice_dims=(0,),
                                       start_index_map=(0,)),
            slice_sizes=(1,),
            mode=lax.GatherScatterMode.PROMISE_IN_BOUNDS)
        v = op(v, pv)
    return v


def _lane_pack(vals, lane):
    out = jnp.zeros((16,), vals[0].dtype)
    for i, v in enumerate(vals):
        out = jnp.where(lane == i, v, out)
    return out


def _sc_body(mask_hbm, out_hbm, part_hbm, buf0, buf1, colcnt, tmp8, tb, tmpf,
             sem0, sem1):
    c = lax.axis_index("c")
    s = lax.axis_index("s")
    wid = c * 16 + s
    b = wid // _SPI
    p = wid % _SPI
    row_base = b * _H + p * _ROWS_PER_SUB
    y_base = p * _ROWS_PER_SUB

    zero16 = jnp.zeros((16,), jnp.int32)
    one_i = jnp.int32(1)
    zero_i = jnp.int32(0)
    for j in range(_NJ):
        colcnt[pl.ds(j * 16, 16)] = zero16

    bufs = (buf0, buf1)
    sems = (sem0, sem1)
    cps = [None, None]
    cps[0] = pltpu.async_copy(mask_hbm.at[pl.ds(row_base, _CHUNK)], buf0, sem0)

    big_v = jnp.full((16,), _BIG, jnp.int32)
    neg1_v = jnp.full((16,), -1, jnp.int32)
    carry = (zero16, big_v, neg1_v)

    for ch in range(_NCH):
        cur = ch % 2
        nxt = (ch + 1) % 2
        if ch + 1 < _NCH:
            cps[nxt] = pltpu.async_copy(
                mask_hbm.at[pl.ds(row_base + (ch + 1) * _CHUNK, _CHUNK)],
                bufs[nxt], sems[nxt])
        cps[cur].wait()
        buf = bufs[cur]
        y0 = y_base + ch * _CHUNK

        def row_body(r, carry, buf=buf, y0=y0):
            sum_y, y_min, y_max = carry

            @plsc.parallel_loop(0, _W, step=16, unroll=8, carry=zero16)
            def rowcnt(off, rc):
                v = buf[r, pl.ds(off, 16)]
                sel = jnp.where(v > 0.5, one_i, zero_i)
                plsc.addupdate(colcnt.at[pl.ds(off, 16)], sel)
                return rc + sel

            yv = jnp.broadcast_to(y0 + r, (16,)).astype(jnp.int32)
            any_ = rowcnt > 0
            sum_y = sum_y + yv * rowcnt
            y_min = jnp.minimum(y_min, jnp.where(any_, yv, big_v))
            y_max = jnp.where(any_, yv, y_max)
            return (sum_y, y_min, y_max)

        carry = lax.fori_loop(0, _CHUNK, row_body, carry)

    sum_y_v, y_min_v, y_max_v = carry

    lane = lax.iota(jnp.int32, 16)
    cnt_v = zero16
    sum_x_v = zero16
    x_min_v = big_v
    x_max_v = neg1_v
    for j in range(_NJ):
        cc = colcnt[pl.ds(j * 16, 16)]
        xv = lane + (j * 16)
        cnt_v = cnt_v + cc
        sum_x_v = sum_x_v + xv * cc
        colany = cc > 0
        x_min_v = jnp.minimum(x_min_v, jnp.where(colany, xv, big_v))
        x_max_v = jnp.maximum(x_max_v, jnp.where(colany, xv, neg1_v))

    cnt_r = _lane_allreduce(cnt_v, jnp.add, lane)
    sy_r = _lane_allreduce(sum_y_v, jnp.add, lane)
    sx_r = _lane_allreduce(sum_x_v, jnp.add, lane)
    ymin_r = _lane_allreduce(y_min_v, jnp.minimum, lane)
    ymax_r = _lane_allreduce(y_max_v, jnp.maximum, lane)
    xmin_r = _lane_allreduce(x_min_v, jnp.minimum, lane)
    xmax_r = _lane_allreduce(x_max_v, jnp.maximum, lane)

    tmp8[0, :] = cnt_r
    tmp8[1, :] = sy_r
    tmp8[2, :] = sx_r
    tmp8[3, :] = ymin_r
    tmp8[4, :] = ymax_r
    tmp8[5, :] = xmin_r
    tmp8[6, :] = xmax_r
    tmp8[7, :] = zero16

    pltpu.sync_copy(tmp8, part_hbm.at[wid])
    plsc.subcore_barrier()

    @pl.when(p == 0)
    def _():
        count_v, sy_v, sx_v = cnt_r, sy_r, sx_r
        ymin_v, ymax_v, xmin_v, xmax_v = ymin_r, ymax_r, xmin_r, xmax_r
        pltpu.sync_copy(part_hbm.at[pl.ds(wid + 1, _SPI - 1)], tb)
        for q in range(_SPI - 1):
            count_v = count_v + tb[q, 0, :]
            sy_v = sy_v + tb[q, 1, :]
            sx_v = sx_v + tb[q, 2, :]
            ymin_v = jnp.minimum(ymin_v, tb[q, 3, :])
            ymax_v = jnp.maximum(ymax_v, tb[q, 4, :])
            xmin_v = jnp.minimum(xmin_v, tb[q, 5, :])
            xmax_v = jnp.maximum(xmax_v, tb[q, 6, :])

        height = ymax_v - ymin_v + 1
        width = xmax_v - xmin_v + 1
        size = jnp.maximum(height, width)
        cy_i = lax.shift_right_arithmetic(ymin_v + ymax_v, 1)
        cx_i = lax.shift_right_arithmetic(xmin_v + xmax_v, 1)
        half_sz = lax.shift_right_arithmetic(size, 1)
        y1 = jnp.maximum(0, cy_i - half_sz)
        x1 = jnp.maximum(0, cx_i - half_sz)
        y2 = jnp.minimum(_H, cy_i + half_sz)
        x2 = jnp.minimum(_W, cx_i + half_sz)

        empty = count_v == 0
        denom = jnp.maximum(count_v.astype(jnp.float32), 1.0)
        cy_f = sy_v.astype(jnp.float32) / denom
        cx_f = sx_v.astype(jnp.float32) / denom
        cy_f = jnp.where(empty, jnp.float32(_H // 2), cy_f)
        cx_f = jnp.where(empty, jnp.float32(_W // 2), cx_f)
        size_f = jnp.where(empty, min(_H, _W) // 2, size).astype(jnp.float32)
        y1_f = jnp.where(empty, _H // 4, y1).astype(jnp.float32)
        x1_f = jnp.where(empty, _W // 4, x1).astype(jnp.float32)
        y2_f = jnp.where(empty, 3 * _H // 4, y2).astype(jnp.float32)
        x2_f = jnp.where(empty, 3 * _W // 4, x2).astype(jnp.float32)

        outv = _lane_pack([cy_f, cx_f, size_f, y1_f, x1_f, y2_f, x2_f], lane)
        tmpf[...] = outv
        pltpu.sync_copy(tmpf, out_hbm.at[b])


_sc_kernel = functools.partial(
    pl.kernel,
    mesh=plsc.VectorSubcoreMesh(core_axis_name="c", subcore_axis_name="s"),
    out_type=(
        jax.ShapeDtypeStruct((_NSC, 16), jnp.float32),
        jax.ShapeDtypeStruct((32, 8, 16), jnp.int32),
    ),
    scratch_types=[
        pltpu.VMEM((_CHUNK, _W), jnp.float32),
        pltpu.VMEM((_CHUNK, _W), jnp.float32),
        pltpu.VMEM((_W,), jnp.int32),
        pltpu.VMEM((8, 16), jnp.int32),
        pltpu.VMEM((_SPI - 1, 8, 16), jnp.int32),
        pltpu.VMEM((16,), jnp.float32),
        pltpu.SemaphoreType.DMA,
        pltpu.SemaphoreType.DMA,
    ],
)(_sc_body)


def _tc_body(mask_ref, stats_ref):
    v = mask_ref[0]
    mb = (v > 0.5).astype(jnp.bfloat16)

    ones_w = jnp.ones((_W, 1), jnp.bfloat16)
    rowcount = jax.lax.dot_general(
        mb, ones_w, (((1,), (0,)), ((), ())),
        preferred_element_type=jnp.float32)[:, 0]
    colcount = jnp.sum(mb.astype(jnp.float32), axis=0)

    yidx = lax.iota(jnp.int32, _H).astype(jnp.float32)
    xidx = lax.iota(jnp.int32, _W).astype(jnp.float32)
    count = jnp.sum(rowcount)
    sum_y = jnp.sum(yidx * rowcount)
    sum_x = jnp.sum(xidx * colcount)

    bigf = jnp.float32(_BIG)
    y_min = jnp.min(jnp.where(rowcount > 0.0, yidx, bigf)).astype(jnp.int32)
    y_max = jnp.max(jnp.where(rowcount > 0.0, yidx, -1.0)).astype(jnp.int32)
    x_min = jnp.min(jnp.where(colcount > 0.0, xidx, bigf)).astype(jnp.int32)
    x_max = jnp.max(jnp.where(colcount > 0.0, xidx, -1.0)).astype(jnp.int32)

    denom = jnp.maximum(count, 1.0)
    center_y = sum_y / denom
    center_x = sum_x / denom

    height = y_max - y_min + 1
    width = x_max - x_min + 1
    size = jnp.maximum(height, width)

    cy_i = (y_min + y_max) // 2
    cx_i = (x_min + x_max) // 2
    half = size // 2
    y1 = jnp.maximum(0, cy_i - half)
    x1 = jnp.maximum(0, cx_i - half)
    y2 = jnp.minimum(_H, cy_i + half)
    x2 = jnp.minimum(_W, cx_i + half)

    empty = count == 0.0
    center_y = jnp.where(empty, jnp.float32(_H // 2), center_y)
    center_x = jnp.where(empty, jnp.float32(_W // 2), center_x)
    size_out = jnp.where(empty, jnp.int32(min(_H, _W) // 2), size)
    y1 = jnp.where(empty, jnp.int32(_H // 4), y1)
    x1 = jnp.where(empty, jnp.int32(_W // 4), x1)
    y2 = jnp.where(empty, jnp.int32(3 * _H // 4), y2)
    x2 = jnp.where(empty, jnp.int32(3 * _W // 4), x2)

    stats_ref[0, 0, 0] = center_y
    stats_ref[0, 0, 1] = center_x
    stats_ref[0, 0, 2] = size_out.astype(jnp.float32)
    stats_ref[0, 0, 3] = y1.astype(jnp.float32)
    stats_ref[0, 0, 4] = x1.astype(jnp.float32)
    stats_ref[0, 0, 5] = y2.astype(jnp.float32)
    stats_ref[0, 0, 6] = x2.astype(jnp.float32)
    for k in range(7, 16):
        stats_ref[0, 0, k] = jnp.float32(0)


def _tc_kernel(m3):
    return pl.pallas_call(
        _tc_body,
        grid=(_NTC,),
        in_specs=[pl.BlockSpec((1, _H, _W), lambda b: (b + _NSC, 0, 0))],
        out_specs=[
            pl.BlockSpec((1, 1, 16), lambda b: (b, 0, 0),
                         memory_space=pltpu.SMEM),
        ],
        out_shape=[
            jax.ShapeDtypeStruct((_NTC, 1, 16), jnp.float32),
        ],
        compiler_params=pltpu.CompilerParams(
            dimension_semantics=("arbitrary",),
        ),
    )(m3)


@jax.jit
def kernel(mask):
    B = mask.shape[0]
    m3 = mask.reshape(B, _H, _W)
    m2 = m3.reshape(B * _H, _W)

    sc_stats, _ = _sc_kernel(m2)
    (tc_stats,) = _tc_kernel(m3)

    stats = jnp.concatenate([sc_stats, tc_stats.reshape(_NTC, 16)], axis=0)
    centers = stats[:, :2]
    sizes = stats[:, 2].astype(jnp.int32)
    bboxes = stats[:, 3:7].astype(jnp.int32)
    return centers, sizes, bboxes


# speedup vs baseline: 2.5269x; 1.7168x over previous
import jax
import jax.numpy as jnp
from jax import lax
from jax.experimental import pallas as pl
from jax.experimental.pallas import tpu as pltpu

_H = 512
_W = 512
_B = 16
_BIG = _H + _W
_NSC = 0
_NTC = _B


def _tc_body(mask_ref, stats_ref):
    v = mask_ref[0]
    mb = (v > 0.5).astype(jnp.bfloat16)

    ones_w = jnp.ones((_W, 1), jnp.bfloat16)
    rowcount = jax.lax.dot_general(
        mb, ones_w, (((1,), (0,)), ((), ())),
        preferred_element_type=jnp.float32)[:, 0]
    colcount = jnp.sum(mb.astype(jnp.float32), axis=0)

    yidx = lax.iota(jnp.int32, _H).astype(jnp.float32)
    xidx = lax.iota(jnp.int32, _W).astype(jnp.float32)
    count = jnp.sum(rowcount)
    sum_y = jnp.sum(yidx * rowcount)
    sum_x = jnp.sum(xidx * colcount)

    bigf = jnp.float32(_BIG)
    y_min = jnp.min(jnp.where(rowcount > 0.0, yidx, bigf)).astype(jnp.int32)
    y_max = jnp.max(jnp.where(rowcount > 0.0, yidx, -1.0)).astype(jnp.int32)
    x_min = jnp.min(jnp.where(colcount > 0.0, xidx, bigf)).astype(jnp.int32)
    x_max = jnp.max(jnp.where(colcount > 0.0, xidx, -1.0)).astype(jnp.int32)

    denom = jnp.maximum(count, 1.0)
    center_y = sum_y / denom
    center_x = sum_x / denom

    height = y_max - y_min + 1
    width = x_max - x_min + 1
    size = jnp.maximum(height, width)

    cy_i = (y_min + y_max) // 2
    cx_i = (x_min + x_max) // 2
    half = size // 2
    y1 = jnp.maximum(0, cy_i - half)
    x1 = jnp.maximum(0, cx_i - half)
    y2 = jnp.minimum(_H, cy_i + half)
    x2 = jnp.minimum(_W, cx_i + half)

    empty = count == 0.0
    center_y = jnp.where(empty, jnp.float32(_H // 2), center_y)
    center_x = jnp.where(empty, jnp.float32(_W // 2), center_x)
    size_out = jnp.where(empty, jnp.int32(min(_H, _W) // 2), size)
    y1 = jnp.where(empty, jnp.int32(_H // 4), y1)
    x1 = jnp.where(empty, jnp.int32(_W // 4), x1)
    y2 = jnp.where(empty, jnp.int32(3 * _H // 4), y2)
    x2 = jnp.where(empty, jnp.int32(3 * _W // 4), x2)

    stats_ref[0, 0, 0] = center_y
    stats_ref[0, 0, 1] = center_x
    stats_ref[0, 0, 2] = size_out.astype(jnp.float32)
    stats_ref[0, 0, 3] = y1.astype(jnp.float32)
    stats_ref[0, 0, 4] = x1.astype(jnp.float32)
    stats_ref[0, 0, 5] = y2.astype(jnp.float32)
    stats_ref[0, 0, 6] = x2.astype(jnp.float32)
    for k in range(7, 16):
        stats_ref[0, 0, k] = jnp.float32(0)


def _tc_kernel(m3):
    return pl.pallas_call(
        _tc_body,
        grid=(_NTC,),
        in_specs=[pl.BlockSpec((1, _H, _W), lambda b: (b + _NSC, 0, 0))],
        out_specs=[
            pl.BlockSpec((1, 1, 16), lambda b: (b, 0, 0),
                         memory_space=pltpu.SMEM),
        ],
        out_shape=[
            jax.ShapeDtypeStruct((_NTC, 1, 16), jnp.float32),
        ],
        compiler_params=pltpu.CompilerParams(
            dimension_semantics=("arbitrary",),
        ),
    )(m3)


@jax.jit
def kernel(mask):
    B = mask.shape[0]
    m3 = mask.reshape(B, _H, _W)
    (tc_stats,) = _tc_kernel(m3)
    stats = tc_stats.reshape(B, 16)
    centers = stats[:, :2]
    sizes = stats[:, 2].astype(jnp.int32)
    bboxes = stats[:, 3:7].astype(jnp.int32)
    return centers, sizes, bboxes
